# baseline (speedup 1.0000x reference)
import functools

import jax
import jax.numpy as jnp
from jax import lax
from jax.experimental import pallas as pl
from jax.experimental.pallas import tpu as pltpu
from jax.experimental.pallas import tpu_sc as plsc

_LANES = 16
_NC = 2
_NS = 16
_CHUNK = 80
_HALF = 128


def _ceil_to(x, m):
    return (x + m - 1) // m * m


@functools.cache
def _make_segsum(cpt, cpt_pad, n_pad, n_wpad):
    zrows = n_pad // _NS
    wpt = n_wpad // _NS
    cpt_r = _ceil_to(cpt, 4)
    mesh = plsc.VectorSubcoreMesh(core_axis_name="c", subcore_axis_name="s",
                                  num_cores=_NC, num_subcores=_NS)

    def body(vals, gidx, sidx, out, acc, gi, si, rows0, rows1,
             mi0, mi1, mi2, mi3, mg0, mg1):
        c = lax.axis_index("c")
        s = lax.axis_index("s")
        zero = jnp.zeros((_LANES,), jnp.float32)
        semi = (mi0, mi1, mi2, mi3)

        def load_idx(slot, chunk):
            pltpu.async_copy(gidx.at[c, s, chunk], gi.at[slot], semi[slot])
            pltpu.async_copy(sidx.at[s, chunk], si.at[slot], semi[slot])

        def wait_idx(slot):
            pltpu.make_async_copy(gidx.at[c, s, 0], gi.at[slot],
                                  semi[slot]).wait()
            pltpu.make_async_copy(sidx.at[s, 0], si.at[slot],
                                  semi[slot]).wait()

        def gather(slot, rbuf, semg):
            pltpu.async_copy(vals.at[gi.at[slot, 0]], rbuf, semg)

        def wait_gather(rbuf, semg):
            pltpu.make_async_copy(vals.at[gi.at[0, 0]], rbuf, semg).wait()

        load_idx(0, 0)
        load_idx(1, 1)
        load_idx(2, 2)
        load_idx(3, 3)

        @pl.loop(0, _CHUNK)
        def _zero_rows(r):
            for g in range(_HALF // _LANES):
                rows0[r, pl.ds(g * _LANES, _LANES)] = zero

        off = 0
        while off < zrows:
            blk = min(_CHUNK, zrows - off)
            pltpu.sync_copy(rows0.at[pl.ds(0, blk)],
                            acc.at[pl.ds(s * zrows + off, blk)])
            off += blk

        wait_idx(0)
        gather(0, rows0, mg0)
        wait_idx(1)
        gather(1, rows1, mg1)
        plsc.subcore_barrier()

        @pl.loop(0, cpt_r // 4)
        def _edges(jj):
            b = 4 * jj
            for k in range(4):
                rbuf = rows0 if k % 2 == 0 else rows1
                semg = mg0 if k % 2 == 0 else mg1
                wait_gather(rbuf, semg)
                pltpu.sync_copy(rbuf, acc.at[si.at[k, 0]], add=True)
                load_idx(k, b + 4 + k)
                kn = (k + 2) % 4
                wait_idx(kn)
                gather(kn, rbuf, semg)

        wait_gather(rows0, mg0)
        wait_gather(rows1, mg1)
        wait_idx(2)
        wait_idx(3)
        plsc.subcore_barrier()
        pltpu.sync_copy(acc.at[pl.ds(s * wpt, wpt)],
                        out.at[c, pl.ds(s * wpt, wpt)])

    return pl.kernel(
        body,
        out_type=jax.ShapeDtypeStruct((_NC, n_wpad, _HALF), jnp.float32),
        mesh=mesh,
        scratch_types=[
            pltpu.VMEM_SHARED((n_pad, _HALF), jnp.float32),
            pltpu.VMEM((4, 1, _CHUNK), jnp.int32),
            pltpu.VMEM((4, 1, _CHUNK), jnp.int32),
            pltpu.VMEM((_CHUNK, _HALF), jnp.float32),
            pltpu.VMEM((_CHUNK, _HALF), jnp.float32),
            pltpu.SemaphoreType.DMA,
            pltpu.SemaphoreType.DMA,
            pltpu.SemaphoreType.DMA,
            pltpu.SemaphoreType.DMA,
            pltpu.SemaphoreType.DMA,
            pltpu.SemaphoreType.DMA,
        ],
    )


@functools.cache
def _make_degree(cpt, cpt_pad, n_pad, n_wpad):
    zrows = n_pad // _NS
    wpt = n_wpad // _NS
    mesh = plsc.VectorSubcoreMesh(core_axis_name="c", subcore_axis_name="s",
                                  num_cores=_NC, num_subcores=_NS)

    def body(didx, out, acc, islab, buf, sem):
        c = lax.axis_index("c")
        s = lax.axis_index("s")

        pltpu.sync_copy(didx.at[c, s, pl.ds(0, cpt)],
                        islab.at[pl.ds(0, cpt)])

        @pl.loop(0, _CHUNK)
        def _zero(r):
            for g in range(_HALF // _LANES):
                buf[r, pl.ds(g * _LANES, _LANES)] = jnp.zeros((_LANES,),
                                                              jnp.float32)

        off = 0
        while off < zrows:
            blk = min(_CHUNK, zrows - off)
            pltpu.sync_copy(buf.at[pl.ds(0, blk)],
                            acc.at[pl.ds(s * zrows + off, blk)])
            off += blk
        plsc.subcore_barrier()

        @pl.loop(0, _CHUNK)
        def _ones(r):
            for g in range(_HALF // _LANES):
                buf[r, pl.ds(g * _LANES, _LANES)] = jnp.ones((_LANES,),
                                                             jnp.float32)

        @pl.loop(0, cpt)
        def _edges(j):
            pltpu.async_copy(buf, acc.at[islab.at[j, 0]], sem, add=True)

        @pl.loop(0, cpt)
        def _drain(j):
            pltpu.make_async_copy(buf, acc.at[islab.at[0, 0]], sem).wait()

        plsc.subcore_barrier()
        pltpu.sync_copy(acc.at[pl.ds(s * wpt, wpt)],
                        out.at[c, pl.ds(s * wpt, wpt)])

    return pl.kernel(
        body,
        out_type=jax.ShapeDtypeStruct((_NC, n_wpad, _HALF), jnp.float32),
        mesh=mesh,
        scratch_types=[
            pltpu.VMEM_SHARED((n_pad, _HALF), jnp.float32),
            pltpu.VMEM((cpt, 1, _CHUNK), jnp.int32),
            pltpu.VMEM((_CHUNK, _HALF), jnp.float32),
            pltpu.SemaphoreType.DMA,
        ],
    )


def _pick_bm(n):
    for bm in (512, 400, 256, 200, 128, 80, 40, 8):
        if n % bm == 0:
            return bm
    return n


def _mm(x, w, b, relu):
    n, k = x.shape
    m = w.shape[1]
    bm = _pick_bm(n)

    def body(x_ref, w_ref, b_ref, o_ref):
        acc = jnp.dot(x_ref[...], w_ref[...],
                      preferred_element_type=jnp.float32,
                      precision=lax.Precision.HIGHEST)
        acc = acc + b_ref[...]
        o_ref[...] = jnp.maximum(acc, 0.0) if relu else acc

    return pl.pallas_call(
        body,
        grid=(n // bm,),
        in_specs=[
            pl.BlockSpec((bm, k), lambda i: (i, 0)),
            pl.BlockSpec((k, m), lambda i: (0, 0)),
            pl.BlockSpec((1, m), lambda i: (0, 0)),
        ],
        out_specs=pl.BlockSpec((bm, m), lambda i: (i, 0)),
        out_shape=jax.ShapeDtypeStruct((n, m), jnp.float32),
    )(x, w, b.reshape(1, m))


def _layer_mm(aggsum, deg, h, w_a0, w_a1, w_h, b_all):
    n, d = h.shape
    bm = _pick_bm(n)

    def body(a_ref, c_ref, h_ref, w0_ref, w1_ref, wh_ref, b_ref,
             hn_ref, hg_ref, hq_ref):
        ic = 1.0 / jnp.maximum(c_ref[0][:, 0:1], 1.0)
        acc = (jnp.dot(a_ref[0] * ic, w0_ref[...],
                       preferred_element_type=jnp.float32,
                       precision=lax.Precision.HIGHEST)
               + jnp.dot(a_ref[1] * ic, w1_ref[...],
                         preferred_element_type=jnp.float32,
                         precision=lax.Precision.HIGHEST)
               + jnp.dot(h_ref[...], wh_ref[...],
                         preferred_element_type=jnp.float32,
                         precision=lax.Precision.HIGHEST)
               + b_ref[...])
        hn = jnp.maximum(acc[:, :d], 0.0)
        hg = jnp.maximum(acc[:, d:], 0.0)
        hn_ref[...] = hn
        hg_ref[...] = hg
        hq_ref[...] = hg * hg

    sds = jax.ShapeDtypeStruct((n, d), jnp.float32)
    return pl.pallas_call(
        body,
        grid=(n // bm,),
        in_specs=[
            pl.BlockSpec((_NC, bm, _HALF), lambda i: (0, i, 0)),
            pl.BlockSpec((1, bm, _HALF), lambda i: (1, i, 0)),
            pl.BlockSpec((bm, d), lambda i: (i, 0)),
            pl.BlockSpec((_HALF, 2 * d), lambda i: (0, 0)),
            pl.BlockSpec((_HALF, 2 * d), lambda i: (0, 0)),
            pl.BlockSpec((d, 2 * d), lambda i: (0, 0)),
            pl.BlockSpec((1, 2 * d), lambda i: (0, 0)),
        ],
        out_specs=[
            pl.BlockSpec((bm, d), lambda i: (i, 0)),
            pl.BlockSpec((bm, d), lambda i: (i, 0)),
            pl.BlockSpec((bm, d), lambda i: (i, 0)),
        ],
        out_shape=[sds, sds, sds],
    )(aggsum, deg, h, w_a0, w_a1, w_h, b_all.reshape(1, 2 * d))


def _gate(h, hn, hg, s1, s2, deg):
    n, d = h.shape
    bm = _pick_bm(n)

    def body(h_ref, hn_ref, hg_ref, s1_ref, s2_ref, d_ref, o_ref):
        dd = d_ref[0][:, 0:1]
        invd = 1.0 / jnp.maximum(dd, 1.0)
        hgv = hg_ref[...]
        s1v = jnp.concatenate([s1_ref[0], s1_ref[1]], axis=1)
        s2v = jnp.concatenate([s2_ref[0], s2_ref[1]], axis=1)
        num = dd * hgv * hgv - 2.0 * hgv * s1v + s2v
        tau = jnp.tanh(num * invd)
        hv = h_ref[...]
        o_ref[...] = hv + tau * (hn_ref[...] - hv)

    return pl.pallas_call(
        body,
        grid=(n // bm,),
        in_specs=[
            pl.BlockSpec((bm, d), lambda i: (i, 0)),
            pl.BlockSpec((bm, d), lambda i: (i, 0)),
            pl.BlockSpec((bm, d), lambda i: (i, 0)),
            pl.BlockSpec((_NC, bm, _HALF), lambda i: (0, i, 0)),
            pl.BlockSpec((_NC, bm, _HALF), lambda i: (0, i, 0)),
            pl.BlockSpec((1, bm, _HALF), lambda i: (0, i, 0)),
        ],
        out_specs=pl.BlockSpec((bm, d), lambda i: (i, 0)),
        out_shape=jax.ShapeDtypeStruct((n, d), jnp.float32),
    )(h, hn, hg, s1, s2, deg)


def kernel(X, edge_index, enc_W, enc_b, dec_W, dec_b,
           conv_Wl, conv_bl, conv_Wr, gg_Wl, gg_bl, gg_Wr):
    n = X.shape[0]
    e = edge_index.shape[1]
    d = conv_Wl.shape[0]

    n_chunks = _ceil_to((e + _CHUNK - 1) // _CHUNK, _NS)
    e_pad = n_chunks * _CHUNK
    cpt = n_chunks // _NS
    cpt_pad = _ceil_to(cpt, 4) + 4
    n_pad = _ceil_to(n + 1, _NS * 8)
    n_wpad = _ceil_to(n, _NS * 8)

    src = edge_index[0]
    dst = edge_index[1]
    padz = jnp.zeros((e_pad - e,), jnp.int32)
    padt = jnp.full((e_pad - e,), n, jnp.int32)

    def chunked(arr, pad_val):
        a = arr.reshape(_NS, cpt, _CHUNK)
        a = jnp.pad(a, ((0, 0), (0, cpt_pad - cpt), (0, 0)),
                    constant_values=pad_val)
        return a.reshape(_NS, cpt_pad, 1, _CHUNK)

    srcg = jnp.concatenate([src, padz])
    dstg = jnp.concatenate([dst, padz])
    srcs = chunked(jnp.concatenate([src, padt]), n)
    dsts = chunked(jnp.concatenate([dst, padt]), n)
    gsrc = jnp.stack([chunked(2 * srcg, 0), chunked(2 * srcg + 1, 0)])
    gdst = jnp.stack([chunked(2 * dstg, 0), chunked(2 * dstg + 1, 0)])
    didx = jnp.stack([srcs, dsts])

    seg = _make_segsum(cpt, cpt_pad, n_pad, n_wpad)
    degk = _make_degree(cpt, cpt_pad, n_pad, n_wpad)

    deg = degk(didx)
    H = _mm(X, enc_W, enc_b, True)

    w_conv = jnp.concatenate([conv_Wl, gg_Wl], axis=1)
    w_a0 = w_conv[:_HALF]
    w_a1 = w_conv[_HALF:]
    w_h = jnp.concatenate([conv_Wr, gg_Wr], axis=1)
    b_all = jnp.concatenate([conv_bl, gg_bl])

    prev = deg
    for _ in range(3):
        hr, _ = lax.optimization_barrier((H.reshape(2 * n, _HALF), prev))
        aggsum = seg(hr, gsrc, dsts)
        hn, hg, hq = _layer_mm(aggsum, deg, H, w_a0, w_a1, w_h, b_all)
        s1 = seg(hg.reshape(2 * n, _HALF), gdst, srcs)
        hqr, _ = lax.optimization_barrier((hq.reshape(2 * n, _HALF), s1))
        s2 = seg(hqr, gdst, srcs)
        prev = s2
        H = _gate(H, hn, hg, s1, s2, deg)

    return _mm(H, dec_W, dec_b, False)


# speedup vs baseline: 2.1010x; 2.1010x over previous
import functools

import jax
import jax.numpy as jnp
from jax import lax
from jax.experimental import pallas as pl
from jax.experimental.pallas import tpu as pltpu
from jax.experimental.pallas import tpu_sc as plsc

_LANES = 16
_NC = 2
_NS = 16
_CHUNK = 128
_GRP = 32
_CHUNK_D = 96
_HALF = 128


def _ceil_to(x, m):
    return (x + m - 1) // m * m


@functools.cache
def _make_segsum(cpt, cpt_pad, n_pad, n_wpad):
    zrows = n_pad // _NS
    wpt = n_wpad // _NS
    mesh = plsc.VectorSubcoreMesh(core_axis_name="c", subcore_axis_name="s",
                                  num_cores=_NC, num_subcores=_NS)
    groups = []
    base = 0
    while base < cpt:
        groups.append((base, min(_GRP, cpt - base)))
        base += _GRP

    def body(idx, vals, out, acc, islab, rows):
        c = lax.axis_index("c")
        s = lax.axis_index("s")
        zero = jnp.zeros((_LANES,), jnp.float32)

        @pl.loop(0, _CHUNK)
        def _zero_rows(r):
            for g in range(_HALF // _LANES):
                rows[r, pl.ds(g * _LANES, _LANES)] = zero

        off = 0
        while off < zrows:
            blk = min(_CHUNK, zrows - off)
            pltpu.sync_copy(rows.at[pl.ds(0, blk)],
                            acc.at[pl.ds(s * zrows + off, blk)])
            off += blk
        plsc.subcore_barrier()

        for g0, glen in groups:
            pltpu.sync_copy(idx.at[c, s, pl.ds(g0, glen)],
                            islab.at[pl.ds(0, glen)])

            @pl.loop(0, glen)
            def _edges(j):
                pltpu.sync_copy(vals.at[islab.at[j, 0]], rows)
                pltpu.sync_copy(rows, acc.at[islab.at[j, 1]], add=True)

        plsc.subcore_barrier()
        pltpu.sync_copy(acc.at[pl.ds(s * wpt, wpt)],
                        out.at[c, pl.ds(s * wpt, wpt)])

    return pl.kernel(
        body,
        out_type=jax.ShapeDtypeStruct((_NC, n_wpad, _HALF), jnp.float32),
        mesh=mesh,
        scratch_types=[
            pltpu.VMEM_SHARED((n_pad, _HALF), jnp.float32),
            pltpu.VMEM((_GRP, 2, _CHUNK), jnp.int32),
            pltpu.VMEM((_CHUNK, _HALF), jnp.float32),
        ],
    )


@functools.cache
def _make_degree(cpt, n_pad, n_wpad):
    zrows = n_pad // _NS
    wpt = n_wpad // _NS
    mesh = plsc.VectorSubcoreMesh(core_axis_name="c", subcore_axis_name="s",
                                  num_cores=_NC, num_subcores=_NS)

    def body(didx, out, acc, islab, buf, sem):
        c = lax.axis_index("c")
        s = lax.axis_index("s")

        pltpu.sync_copy(didx.at[c, s], islab)

        @pl.loop(0, _CHUNK_D)
        def _zero(r):
            for g in range(_HALF // _LANES):
                buf[r, pl.ds(g * _LANES, _LANES)] = jnp.zeros((_LANES,),
                                                              jnp.float32)

        off = 0
        while off < zrows:
            blk = min(_CHUNK_D, zrows - off)
            pltpu.sync_copy(buf.at[pl.ds(0, blk)],
                            acc.at[pl.ds(s * zrows + off, blk)])
            off += blk
        plsc.subcore_barrier()

        @pl.loop(0, _CHUNK_D)
        def _ones(r):
            for g in range(_HALF // _LANES):
                buf[r, pl.ds(g * _LANES, _LANES)] = jnp.ones((_LANES,),
                                                             jnp.float32)

        @pl.loop(0, cpt)
        def _edges(j):
            pltpu.async_copy(buf, acc.at[islab.at[j, 0]], sem, add=True)

        @pl.loop(0, cpt)
        def _drain(j):
            pltpu.make_async_copy(buf, acc.at[islab.at[0, 0]], sem).wait()

        plsc.subcore_barrier()
        pltpu.sync_copy(acc.at[pl.ds(s * wpt, wpt)],
                        out.at[c, pl.ds(s * wpt, wpt)])

    return pl.kernel(
        body,
        out_type=jax.ShapeDtypeStruct((_NC, n_wpad, _HALF), jnp.float32),
        mesh=mesh,
        scratch_types=[
            pltpu.VMEM_SHARED((n_pad, _HALF), jnp.float32),
            pltpu.VMEM((cpt, 1, _CHUNK_D), jnp.int32),
            pltpu.VMEM((_CHUNK_D, _HALF), jnp.float32),
            pltpu.SemaphoreType.DMA,
        ],
    )


def _pick_bm(n):
    for bm in (512, 400, 256, 200, 128, 80, 40, 8):
        if n % bm == 0:
            return bm
    return n


def _mm(x, w, b, relu):
    n, k = x.shape
    m = w.shape[1]
    bm = _pick_bm(n)

    def body(x_ref, w_ref, b_ref, o_ref):
        acc = jnp.dot(x_ref[...], w_ref[...],
                      preferred_element_type=jnp.float32,
                      precision=lax.Precision.HIGHEST)
        acc = acc + b_ref[...]
        o_ref[...] = jnp.maximum(acc, 0.0) if relu else acc

    return pl.pallas_call(
        body,
        grid=(n // bm,),
        in_specs=[
            pl.BlockSpec((bm, k), lambda i: (i, 0)),
            pl.BlockSpec((k, m), lambda i: (0, 0)),
            pl.BlockSpec((1, m), lambda i: (0, 0)),
        ],
        out_specs=pl.BlockSpec((bm, m), lambda i: (i, 0)),
        out_shape=jax.ShapeDtypeStruct((n, m), jnp.float32),
    )(x, w, b.reshape(1, m))


def _layer_mm(aggsum, deg, h, w_a0, w_a1, w_h, b_all):
    n, d = h.shape
    bm = _pick_bm(n)

    def body(a_ref, c_ref, h_ref, w0_ref, w1_ref, wh_ref, b_ref,
             hn_ref, hg_ref, hq_ref):
        ic = 1.0 / jnp.maximum(c_ref[0][:, 0:1], 1.0)
        acc = (jnp.dot(a_ref[0] * ic, w0_ref[...],
                       preferred_element_type=jnp.float32,
                       precision=lax.Precision.HIGHEST)
               + jnp.dot(a_ref[1] * ic, w1_ref[...],
                         preferred_element_type=jnp.float32,
                         precision=lax.Precision.HIGHEST)
               + jnp.dot(h_ref[...], wh_ref[...],
                         preferred_element_type=jnp.float32,
                         precision=lax.Precision.HIGHEST)
               + b_ref[...])
        hn = jnp.maximum(acc[:, :d], 0.0)
        hg = jnp.maximum(acc[:, d:], 0.0)
        hn_ref[...] = hn
        hg_ref[...] = hg
        hq_ref[...] = hg * hg

    sds = jax.ShapeDtypeStruct((n, d), jnp.float32)
    return pl.pallas_call(
        body,
        grid=(n // bm,),
        in_specs=[
            pl.BlockSpec((_NC, bm, _HALF), lambda i: (0, i, 0)),
            pl.BlockSpec((1, bm, _HALF), lambda i: (1, i, 0)),
            pl.BlockSpec((bm, d), lambda i: (i, 0)),
            pl.BlockSpec((_HALF, 2 * d), lambda i: (0, 0)),
            pl.BlockSpec((_HALF, 2 * d), lambda i: (0, 0)),
            pl.BlockSpec((d, 2 * d), lambda i: (0, 0)),
            pl.BlockSpec((1, 2 * d), lambda i: (0, 0)),
        ],
        out_specs=[
            pl.BlockSpec((bm, d), lambda i: (i, 0)),
            pl.BlockSpec((bm, d), lambda i: (i, 0)),
            pl.BlockSpec((bm, d), lambda i: (i, 0)),
        ],
        out_shape=[sds, sds, sds],
    )(aggsum, deg, h, w_a0, w_a1, w_h, b_all.reshape(1, 2 * d))


def _gate(h, hn, hg, s1, s2, deg):
    n, d = h.shape
    bm = _pick_bm(n)

    def body(h_ref, hn_ref, hg_ref, s1_ref, s2_ref, d_ref, o_ref):
        dd = d_ref[0][:, 0:1]
        invd = 1.0 / jnp.maximum(dd, 1.0)
        hgv = hg_ref[...]
        s1v = jnp.concatenate([s1_ref[0], s1_ref[1]], axis=1)
        s2v = jnp.concatenate([s2_ref[0], s2_ref[1]], axis=1)
        num = dd * hgv * hgv - 2.0 * hgv * s1v + s2v
        tau = jnp.tanh(num * invd)
        hv = h_ref[...]
        o_ref[...] = hv + tau * (hn_ref[...] - hv)

    return pl.pallas_call(
        body,
        grid=(n // bm,),
        in_specs=[
            pl.BlockSpec((bm, d), lambda i: (i, 0)),
            pl.BlockSpec((bm, d), lambda i: (i, 0)),
            pl.BlockSpec((bm, d), lambda i: (i, 0)),
            pl.BlockSpec((_NC, bm, _HALF), lambda i: (0, i, 0)),
            pl.BlockSpec((_NC, bm, _HALF), lambda i: (0, i, 0)),
            pl.BlockSpec((1, bm, _HALF), lambda i: (0, i, 0)),
        ],
        out_specs=pl.BlockSpec((bm, d), lambda i: (i, 0)),
        out_shape=jax.ShapeDtypeStruct((n, d), jnp.float32),
    )(h, hn, hg, s1, s2, deg)


def kernel(X, edge_index, enc_W, enc_b, dec_W, dec_b,
           conv_Wl, conv_bl, conv_Wr, gg_Wl, gg_bl, gg_Wr):
    n = X.shape[0]
    e = edge_index.shape[1]
    d = conv_Wl.shape[0]

    n_chunks = _ceil_to((e + _CHUNK - 1) // _CHUNK, _NS)
    e_pad = n_chunks * _CHUNK
    cpt = n_chunks // _NS
    n_chunks_d = _ceil_to((e + _CHUNK_D - 1) // _CHUNK_D, _NS)
    e_pad_d = n_chunks_d * _CHUNK_D
    cpt_d = n_chunks_d // _NS
    n_pad = _ceil_to(n + 1, _NS * 8)
    n_wpad = _ceil_to(n, _NS * 8)

    src = edge_index[0]
    dst = edge_index[1]

    def padded(arr, e_to, pad_val):
        return jnp.concatenate(
            [arr, jnp.full((e_to - e,), pad_val, jnp.int32)])

    def combo(g, sc):
        gg = g.reshape(_NS, cpt, 1, _CHUNK)
        ss = sc.reshape(_NS, cpt, 1, _CHUNK)
        return jnp.concatenate([gg, ss], axis=2)

    srcg = padded(src, e_pad, 0)
    dstg = padded(dst, e_pad, 0)
    srcs = padded(src, e_pad, n)
    dsts = padded(dst, e_pad, n)
    iagg = jnp.stack([combo(2 * srcg, dsts), combo(2 * srcg + 1, dsts)])
    igate = jnp.stack([combo(2 * dstg, srcs), combo(2 * dstg + 1, srcs)])
    didx = jnp.stack([
        padded(src, e_pad_d, n).reshape(_NS, cpt_d, 1, _CHUNK_D),
        padded(dst, e_pad_d, n).reshape(_NS, cpt_d, 1, _CHUNK_D),
    ])

    seg = _make_segsum(cpt, 0, n_pad, n_wpad)
    degk = _make_degree(cpt_d, n_pad, n_wpad)

    deg = degk(didx)
    H = _mm(X, enc_W, enc_b, True)

    w_conv = jnp.concatenate([conv_Wl, gg_Wl], axis=1)
    w_a0 = w_conv[:_HALF]
    w_a1 = w_conv[_HALF:]
    w_h = jnp.concatenate([conv_Wr, gg_Wr], axis=1)
    b_all = jnp.concatenate([conv_bl, gg_bl])

    prev = deg
    for _ in range(3):
        hr, _ = lax.optimization_barrier((H.reshape(2 * n, _HALF), prev))
        aggsum = seg(iagg, hr)
        hn, hg, hq = _layer_mm(aggsum, deg, H, w_a0, w_a1, w_h, b_all)
        s1 = seg(igate, hg.reshape(2 * n, _HALF))
        hqr, _ = lax.optimization_barrier((hq.reshape(2 * n, _HALF), s1))
        s2 = seg(igate, hqr)
        prev = s2
        H = _gate(H, hn, hg, s1, s2, deg)

    return _mm(H, dec_W, dec_b, False)
